# Initial kernel scaffold; baseline (speedup 1.0000x reference)
import jax
import jax.numpy as jnp
from jax.experimental import pallas as pl


def kernel(x, train_mask, edge_index_u, edge_weight_u, wenc, conv_w, wdec):
    raise NotImplementedError("write your pallas kernel here")


# speedup vs baseline: 3.8320x; 3.8320x over previous
import functools

import jax
import jax.numpy as jnp
from jax import lax
from jax.experimental import pallas as pl
from jax.experimental.pallas import tpu as pltpu
from jax.experimental.pallas import tpu_sc as plsc

N_USERS = 10000
N_ITEMS = 2000
EMB = 500
EMBP = 512
NCH = 4
CW = 128
E_U = 160000
REG = 0.001

NT = 16
NB = 79
BB = 128
EP = NT * NB * BB
RPT = N_USERS // NT


_RENC = 400


def _encode_body(x_ref, m_ref, w_ref, xt_ref, z4_ref):
    xt = x_ref[...] * m_ref[...]
    xt_ref[...] = xt
    z = jnp.dot(xt, w_ref[...], preferred_element_type=jnp.float32)
    z = jnp.maximum(z, 0.0)
    z4_ref[...] = z.reshape(_RENC, NCH, CW).transpose(1, 0, 2)


def _encode(x, mask, wenc_p):
    return pl.pallas_call(
        _encode_body,
        grid=(N_USERS // _RENC,),
        in_specs=[
            pl.BlockSpec((_RENC, N_ITEMS), lambda i: (i, 0)),
            pl.BlockSpec((_RENC, N_ITEMS), lambda i: (i, 0)),
            pl.BlockSpec((N_ITEMS, EMBP), lambda i: (0, 0)),
        ],
        out_specs=[
            pl.BlockSpec((_RENC, N_ITEMS), lambda i: (i, 0)),
            pl.BlockSpec((NCH, _RENC, CW), lambda i: (0, i, 0)),
        ],
        out_shape=[
            jax.ShapeDtypeStruct((N_USERS, N_ITEMS), jnp.float32),
            jax.ShapeDtypeStruct((NCH, N_USERS, CW), jnp.float32),
        ],
    )(x, mask, wenc_p)


def _w2_body(we_ref, cw_ref, wd_ref, w2_ref, reg_ref):
    w2 = jnp.dot(cw_ref[...], wd_ref[...], preferred_element_type=jnp.float32)
    w2p = jnp.concatenate(
        [w2, jnp.zeros((EMBP - EMB, N_ITEMS), jnp.float32)], axis=0)
    w2_ref[...] = w2p.reshape(NCH, CW, N_ITEMS)
    reg = (jnp.sum(we_ref[...] ** 2) + jnp.sum(cw_ref[...] ** 2)
           + jnp.sum(wd_ref[...] ** 2))
    reg_ref[0, 0] = reg * (REG / 3.0)


def _w2_and_reg(wenc, conv_w, wdec):
    return pl.pallas_call(
        _w2_body,
        in_specs=[
            pl.BlockSpec(memory_space=pltpu.VMEM),
            pl.BlockSpec(memory_space=pltpu.VMEM),
            pl.BlockSpec(memory_space=pltpu.VMEM),
        ],
        out_specs=[
            pl.BlockSpec(memory_space=pltpu.VMEM),
            pl.BlockSpec(memory_space=pltpu.SMEM),
        ],
        out_shape=[
            jax.ShapeDtypeStruct((NCH, CW, N_ITEMS), jnp.float32),
            jax.ShapeDtypeStruct((1, 1), jnp.float32),
        ],
    )(wenc, conv_w, wdec)


def _decode_body(a_ref, w_ref, o_ref):
    a = a_ref[...].transpose(1, 0, 2).reshape(_RENC, EMBP)
    w = w_ref[...].reshape(EMBP, N_ITEMS)
    o_ref[...] = jnp.dot(a, w, preferred_element_type=jnp.float32)


def _decode(agg4, w2):
    return pl.pallas_call(
        _decode_body,
        grid=(N_USERS // _RENC,),
        in_specs=[
            pl.BlockSpec((NCH, _RENC, CW), lambda i: (0, i, 0)),
            pl.BlockSpec((NCH, CW, N_ITEMS), lambda i: (0, 0, 0)),
        ],
        out_specs=pl.BlockSpec((_RENC, N_ITEMS), lambda i: (i, 0)),
        out_shape=jax.ShapeDtypeStruct((N_USERS, N_ITEMS), jnp.float32),
    )(agg4, w2)


def _rsqrt16(p):
    i = plsc.bitcast(p, jnp.int32)
    i = jnp.int32(0x5F3759DF) - lax.shift_right_logical(i, 1)
    y = plsc.bitcast(i, jnp.float32)
    for _ in range(4):
        y = y * (1.5 - 0.5 * p * y * y)
    return y


DEGP = 10240


def _agg_body(src_hbm, dst_hbm, ew_hbm, z4_hbm, agg_hbm,
              src2d, dst2d, ew2d, rows, zvec,
              dsbuf, ddbuf, deg_sh, agg_sh, sem):
    t = lax.axis_index("s")
    c = lax.axis_index("c")
    zero16 = jnp.zeros((16,), jnp.float32)

    pltpu.sync_copy(src_hbm.at[t], src2d)
    pltpu.sync_copy(dst_hbm.at[t], dst2d)
    pltpu.sync_copy(ew_hbm.at[t], ew2d)

    def _zvecb(i, _):
        zvec[pl.ds(i * 16, 16)] = zero16
        return 0
    lax.fori_loop(0, 80, _zvecb, 0)

    @pl.when(t == 0)
    def _():
        for q in range(DEGP // 1280):
            pltpu.sync_copy(zvec, deg_sh.at[pl.ds(q * 1280, 1280)])
    plsc.subcore_barrier()

    def _degb(b, _):
        pltpu.sync_copy(ew2d.at[b], deg_sh.at[dst2d.at[b]], add=True)
        return 0
    lax.fori_loop(0, NB, _degb, 0)
    plsc.subcore_barrier()

    def _normb(b, _):
        pltpu.async_copy(deg_sh.at[src2d.at[b]], dsbuf, sem).wait()
        pltpu.async_copy(deg_sh.at[dst2d.at[b]], ddbuf, sem).wait()
        for k in range(8):
            sl = pl.ds(k * 16, 16)
            p = dsbuf[sl] * ddbuf[sl] + 1e-12
            ew2d[b, sl] = ew2d[b, sl] * _rsqrt16(p)
        return 0
    lax.fori_loop(0, NB, _normb, 0)

    for p in range(2):
        chunk = c * 2 + p
        off = (c * 2 * N_USERS) if p == 0 else N_USERS

        def _adjb(b, _):
            for k in range(8):
                sl = pl.ds(k * 16, 16)
                src2d[b, sl] = src2d[b, sl] + off
            return 0
        lax.fori_loop(0, NB, _adjb, 0)

        def _zrow(i, _):
            for k in range(8):
                rows[i, pl.ds(k * 16, 16)] = zero16
            return 0
        lax.fori_loop(0, BB, _zrow, 0)
        for q in range(5):
            i = q * NT + t
            @pl.when(i < N_USERS // BB)
            def _():
                pltpu.sync_copy(rows, agg_sh.at[pl.ds(i * BB, BB)])
        @pl.when(t == 0)
        def _():
            pltpu.sync_copy(rows.at[pl.ds(0, N_USERS % BB)],
                            agg_sh.at[pl.ds(N_USERS - N_USERS % BB,
                                            N_USERS % BB)])
        plsc.subcore_barrier()

        def _aggb(b, _):
            pltpu.async_copy(z4_hbm.at[src2d.at[b]], rows, sem).wait()

            def _scale(j, _2):
                nv = ew2d[b, pl.ds(j * 16, 16)]
                for l in range(16):
                    nrm = nv[l]
                    for k in range(8):
                        sl = pl.ds(k * 16, 16)
                        rows[j * 16 + l, sl] = rows[j * 16 + l, sl] * nrm
                return 0
            lax.fori_loop(0, BB // 16, _scale, 0)
            pltpu.sync_copy(rows, agg_sh.at[dst2d.at[b]], add=True)
            return 0
        lax.fori_loop(0, NB, _aggb, 0)
        plsc.subcore_barrier()

        pltpu.sync_copy(agg_sh.at[pl.ds(t * RPT, RPT)],
                        agg_hbm.at[chunk * NT + t])
        plsc.subcore_barrier()


def _sc_agg(src3, dst3, ew3, z4flat):
    mesh = plsc.VectorSubcoreMesh(core_axis_name="c", subcore_axis_name="s",
                                  num_cores=2, num_subcores=NT)
    k = pl.kernel(
        _agg_body,
        out_type=jax.ShapeDtypeStruct((NCH * NT, RPT, CW), jnp.float32),
        mesh=mesh,
        compiler_params=pltpu.CompilerParams(needs_layout_passes=False),
        scratch_types=[
            pltpu.VMEM((NB, BB), jnp.int32),
            pltpu.VMEM((NB, BB), jnp.int32),
            pltpu.VMEM((NB, BB), jnp.float32),
            pltpu.VMEM((BB, CW), jnp.float32),
            pltpu.VMEM((1280,), jnp.float32),
            pltpu.VMEM((BB,), jnp.float32),
            pltpu.VMEM((BB,), jnp.float32),
            pltpu.VMEM_SHARED((DEGP,), jnp.float32),
            pltpu.VMEM_SHARED((N_USERS, CW), jnp.float32),
            pltpu.SemaphoreType.DMA,
        ],
    )
    return k(src3, dst3, ew3, z4flat)


def kernel(x, train_mask, edge_index_u, edge_weight_u, wenc, conv_w, wdec):
    src = edge_index_u[0].astype(jnp.int32)
    dst = edge_index_u[1].astype(jnp.int32)
    e = src.shape[0]
    src3 = jnp.pad(src, (0, EP - e)).reshape(NT, NB, BB)
    dst3 = jnp.pad(dst, (0, EP - e)).reshape(NT, NB, BB)
    ew3 = jnp.pad(edge_weight_u, (0, EP - e)).reshape(NT, NB, BB)
    wenc_p = jnp.pad(wenc, ((0, 0), (0, EMBP - EMB)))

    x_train, z4 = _encode(x, train_mask, wenc_p)
    w2, reg = _w2_and_reg(wenc, conv_w, wdec)
    aggf = _sc_agg(src3, dst3, ew3, z4.reshape(NCH * N_USERS, CW))
    pred = _decode(aggf.reshape(NCH, N_USERS, CW), w2)
    return (x_train, pred, reg.reshape(()))


# speedup vs baseline: 4.4819x; 1.1696x over previous
import functools

import jax
import jax.numpy as jnp
from jax import lax
from jax.experimental import pallas as pl
from jax.experimental.pallas import tpu as pltpu
from jax.experimental.pallas import tpu_sc as plsc

N_USERS = 10000
N_ITEMS = 2000
EMB = 500
EMBP = 512
NCH = 4
CW = 128
E_U = 160000
REG = 0.001

NT = 16
NB = 79
BB = 128
HB = 64
EP = NT * NB * BB
RPT = N_USERS // NT


_RENC = 400


def _encode_body(x_ref, m_ref, w_ref, xt_ref, z4_ref):
    xt = x_ref[...] * m_ref[...]
    xt_ref[...] = xt
    z = jnp.dot(xt, w_ref[...], preferred_element_type=jnp.float32)
    z = jnp.maximum(z, 0.0)
    z4_ref[...] = z.reshape(_RENC, NCH, CW).transpose(1, 0, 2)


def _encode(x, mask, wenc_p):
    return pl.pallas_call(
        _encode_body,
        grid=(N_USERS // _RENC,),
        in_specs=[
            pl.BlockSpec((_RENC, N_ITEMS), lambda i: (i, 0)),
            pl.BlockSpec((_RENC, N_ITEMS), lambda i: (i, 0)),
            pl.BlockSpec((N_ITEMS, EMBP), lambda i: (0, 0)),
        ],
        out_specs=[
            pl.BlockSpec((_RENC, N_ITEMS), lambda i: (i, 0)),
            pl.BlockSpec((NCH, _RENC, CW), lambda i: (0, i, 0)),
        ],
        out_shape=[
            jax.ShapeDtypeStruct((N_USERS, N_ITEMS), jnp.float32),
            jax.ShapeDtypeStruct((NCH, N_USERS, CW), jnp.float32),
        ],
    )(x, mask, wenc_p)


def _w2_body(we_ref, cw_ref, wd_ref, w2_ref, reg_ref):
    w2 = jnp.dot(cw_ref[...], wd_ref[...], preferred_element_type=jnp.float32)
    w2p = jnp.concatenate(
        [w2, jnp.zeros((EMBP - EMB, N_ITEMS), jnp.float32)], axis=0)
    w2_ref[...] = w2p.reshape(NCH, CW, N_ITEMS)
    reg = (jnp.sum(we_ref[...] ** 2) + jnp.sum(cw_ref[...] ** 2)
           + jnp.sum(wd_ref[...] ** 2))
    reg_ref[0, 0] = reg * (REG / 3.0)


def _w2_and_reg(wenc, conv_w, wdec):
    return pl.pallas_call(
        _w2_body,
        in_specs=[
            pl.BlockSpec(memory_space=pltpu.VMEM),
            pl.BlockSpec(memory_space=pltpu.VMEM),
            pl.BlockSpec(memory_space=pltpu.VMEM),
        ],
        out_specs=[
            pl.BlockSpec(memory_space=pltpu.VMEM),
            pl.BlockSpec(memory_space=pltpu.SMEM),
        ],
        out_shape=[
            jax.ShapeDtypeStruct((NCH, CW, N_ITEMS), jnp.float32),
            jax.ShapeDtypeStruct((1, 1), jnp.float32),
        ],
    )(wenc, conv_w, wdec)


def _decode_body(a_ref, w_ref, o_ref):
    a = a_ref[...].transpose(1, 0, 2).reshape(_RENC, EMBP)
    w = w_ref[...].reshape(EMBP, N_ITEMS)
    o_ref[...] = jnp.dot(a, w, preferred_element_type=jnp.float32)


def _decode(agg4, w2):
    return pl.pallas_call(
        _decode_body,
        grid=(N_USERS // _RENC,),
        in_specs=[
            pl.BlockSpec((NCH, _RENC, CW), lambda i: (0, i, 0)),
            pl.BlockSpec((NCH, CW, N_ITEMS), lambda i: (0, 0, 0)),
        ],
        out_specs=pl.BlockSpec((_RENC, N_ITEMS), lambda i: (i, 0)),
        out_shape=jax.ShapeDtypeStruct((N_USERS, N_ITEMS), jnp.float32),
    )(agg4, w2)


def _rsqrt16(p):
    i = plsc.bitcast(p, jnp.int32)
    i = jnp.int32(0x5F3759DF) - lax.shift_right_logical(i, 1)
    y = plsc.bitcast(i, jnp.float32)
    for _ in range(4):
        y = y * (1.5 - 0.5 * p * y * y)
    return y


DEGP = 10240


def _agg_body(src_hbm, dst_hbm, ew_hbm, z4_hbm, agg_hbm,
              src2d, dst2d, ew2d, rows, zvec,
              dsbuf, ddbuf, didx0, didx1, deg_sh, agg_sh, sem, sem1):
    t = lax.axis_index("s")
    c = lax.axis_index("c")
    zero16 = jnp.zeros((16,), jnp.float32)

    pltpu.sync_copy(src_hbm.at[t], src2d)
    pltpu.sync_copy(dst_hbm.at[t], dst2d)
    pltpu.sync_copy(ew_hbm.at[t], ew2d)

    def _zvecb(i, _):
        zvec[pl.ds(i * 16, 16)] = zero16
        return 0
    lax.fori_loop(0, 80, _zvecb, 0)

    @pl.when(t == 0)
    def _():
        for q in range(DEGP // 1280):
            pltpu.sync_copy(zvec, deg_sh.at[pl.ds(q * 1280, 1280)])
    plsc.subcore_barrier()

    def _degb(b, _):
        pltpu.sync_copy(ew2d.at[b], deg_sh.at[dst2d.at[b]], add=True)
        return 0
    lax.fori_loop(0, NB, _degb, 0)
    plsc.subcore_barrier()

    def _normb(b, _):
        pltpu.async_copy(deg_sh.at[src2d.at[b]], dsbuf, sem).wait()
        pltpu.async_copy(deg_sh.at[dst2d.at[b]], ddbuf, sem).wait()
        for k in range(BB // 16):
            sl = pl.ds(k * 16, 16)
            p = dsbuf[sl] * ddbuf[sl] + 1e-12
            ew2d[b, sl] = ew2d[b, sl] * _rsqrt16(p)
        return 0
    lax.fori_loop(0, NB, _normb, 0)

    for p in range(2):
        chunk = c * 2 + p
        off = (c * 2 * N_USERS) if p == 0 else N_USERS

        def _adjb(b, _):
            for k in range(BB // 16):
                sl = pl.ds(k * 16, 16)
                src2d[b, sl] = src2d[b, sl] + off
            return 0
        lax.fori_loop(0, NB, _adjb, 0)

        def _zrow(i, _):
            for k in range(8):
                rows[i, pl.ds(k * 16, 16)] = zero16
            return 0
        lax.fori_loop(0, BB, _zrow, 0)
        nchunk = N_USERS // BB
        for q in range((nchunk + NT - 1) // NT):
            i = q * NT + t
            @pl.when(i < nchunk)
            def _():
                pltpu.sync_copy(rows, agg_sh.at[pl.ds(i * BB, BB)])
        @pl.when(t == 0)
        def _():
            pltpu.sync_copy(rows.at[pl.ds(0, N_USERS % BB)],
                            agg_sh.at[pl.ds(N_USERS - N_USERS % BB,
                                            N_USERS % BB)])
        plsc.subcore_barrier()

        half0 = rows.at[pl.ds(0, HB)]
        half1 = rows.at[pl.ds(HB, HB)]

        def _scale_scatter(h, half, didx):
            b = lax.div(h, 2)
            o = lax.rem(h, 2) * HB
            def _scale(j, _2):
                nv = ew2d[b, pl.ds(o + j * 16, 16)]
                for l in range(16):
                    nrm = nv[l]
                    for k in range(8):
                        sl = pl.ds(k * 16, 16)
                        half[j * 16 + l, sl] = half[j * 16 + l, sl] * nrm
                return 0
            lax.fori_loop(0, HB // 16, _scale, 0)
            for q in range(HB // 16):
                didx[pl.ds(q * 16, 16)] = dst2d[b, pl.ds(o + q * 16, 16)]
            pltpu.sync_copy(half, agg_sh.at[didx], add=True)

        def _gather(h, half, sm):
            b = lax.div(h, 2)
            o = lax.rem(h, 2) * HB
            pltpu.async_copy(z4_hbm.at[src2d.at[b, pl.ds(o, HB)]], half, sm)

        nh = 2 * NB
        _gather(0, half0, sem)
        def _pair(hh, _):
            h0 = 2 * hh
            h1 = h0 + 1
            _gather(h1, half1, sem1)
            pltpu.make_async_copy(z4_hbm.at[pl.ds(0, HB)], half0, sem).wait()
            _scale_scatter(h0, half0, didx0)
            @pl.when(hh < nh // 2 - 1)
            def _():
                _gather(h0 + 2, half0, sem)
            pltpu.make_async_copy(z4_hbm.at[pl.ds(0, HB)], half1, sem1).wait()
            _scale_scatter(h1, half1, didx1)
            return 0
        lax.fori_loop(0, nh // 2, _pair, 0)
        plsc.subcore_barrier()

        pltpu.sync_copy(agg_sh.at[pl.ds(t * RPT, RPT)],
                        agg_hbm.at[chunk * NT + t])
        plsc.subcore_barrier()


def _sc_agg(src3, dst3, ew3, z4flat):
    mesh = plsc.VectorSubcoreMesh(core_axis_name="c", subcore_axis_name="s",
                                  num_cores=2, num_subcores=NT)
    k = pl.kernel(
        _agg_body,
        out_type=jax.ShapeDtypeStruct((NCH * NT, RPT, CW), jnp.float32),
        mesh=mesh,
        compiler_params=pltpu.CompilerParams(needs_layout_passes=False),
        scratch_types=[
            pltpu.VMEM((NB, BB), jnp.int32),
            pltpu.VMEM((NB, BB), jnp.int32),
            pltpu.VMEM((NB, BB), jnp.float32),
            pltpu.VMEM((BB, CW), jnp.float32),
            pltpu.VMEM((1280,), jnp.float32),
            pltpu.VMEM((BB,), jnp.float32),
            pltpu.VMEM((BB,), jnp.float32),
            pltpu.VMEM((HB,), jnp.int32),
            pltpu.VMEM((HB,), jnp.int32),
            pltpu.VMEM_SHARED((DEGP,), jnp.float32),
            pltpu.VMEM_SHARED((N_USERS, CW), jnp.float32),
            pltpu.SemaphoreType.DMA,
            pltpu.SemaphoreType.DMA,
        ],
    )
    return k(src3, dst3, ew3, z4flat)


def kernel(x, train_mask, edge_index_u, edge_weight_u, wenc, conv_w, wdec):
    src = edge_index_u[0].astype(jnp.int32)
    dst = edge_index_u[1].astype(jnp.int32)
    e = src.shape[0]
    src3 = jnp.pad(src, (0, EP - e)).reshape(NT, NB, BB)
    dst3 = jnp.pad(dst, (0, EP - e)).reshape(NT, NB, BB)
    ew3 = jnp.pad(edge_weight_u, (0, EP - e)).reshape(NT, NB, BB)
    wenc_p = jnp.pad(wenc, ((0, 0), (0, EMBP - EMB)))

    x_train, z4 = _encode(x, train_mask, wenc_p)
    w2, reg = _w2_and_reg(wenc, conv_w, wdec)
    aggf = _sc_agg(src3, dst3, ew3, z4.reshape(NCH * N_USERS, CW))
    pred = _decode(aggf.reshape(NCH, N_USERS, CW), w2)
    return (x_train, pred, reg.reshape(()))
